# Initial kernel scaffold; baseline (speedup 1.0000x reference)
import jax
import jax.numpy as jnp
from jax.experimental import pallas as pl


def kernel(x, pxyz, pknn, kernel_points, weights, gamma, beta):
    raise NotImplementedError("write your pallas kernel here")


# speedup vs baseline: 1.2990x; 1.2990x over previous
import functools

import jax
import jax.numpy as jnp
from jax import lax
from jax.experimental import pallas as pl
from jax.experimental.pallas import tpu as pltpu
from jax.experimental.pallas import tpu_sc as plsc

KP_EXTENT = 1.2
KS = 15
CIN = 128
COUT = 128
K = 32
N = 10000
NE = N * K
D = CIN + 128

_CHUNK = 80
_NB = 200


def _sc_gather(idx, table):
    info = plsc.get_sparse_core_info()
    nw = info.num_cores * info.num_subcores
    per_w = NE // nw
    steps = per_w // _CHUNK
    mesh = plsc.VectorSubcoreMesh(core_axis_name="c", subcore_axis_name="s")

    @functools.partial(
        pl.kernel,
        mesh=mesh,
        out_type=jax.ShapeDtypeStruct((NE, D), jnp.float32),
        scratch_types=[
            pltpu.VMEM((_CHUNK,), jnp.int32),
            pltpu.VMEM((_CHUNK, D), jnp.float32),
            pltpu.SemaphoreType.DMA,
        ],
    )
    def gather_k(idx_hbm, table_hbm, out_hbm, idx_v, rows_v, sem):
        wid = lax.axis_index("s") * info.num_cores + lax.axis_index("c")
        def body(i, carry):
            base = wid * per_w + i * _CHUNK
            pltpu.sync_copy(idx_hbm.at[pl.ds(base, _CHUNK)], idx_v)
            pltpu.async_copy(table_hbm.at[idx_v], rows_v, sem).wait()
            pltpu.sync_copy(rows_v, out_hbm.at[pl.ds(base, _CHUNK)])
            return carry
        lax.fori_loop(0, steps, body, 0)

    return gather_k(idx, table)


def _conv_body(g_ref, p_ref, ct_ref, w_ref, o_ref):
    g = g_ref[...]
    gf = g[:, :CIN]
    gc = g[:, CIN:CIN + 16]
    p = p_ref[...]
    pe = jnp.broadcast_to(p[:, None, :], (_NB, K, 16)).reshape(_NB * K, 16)
    e = gc - pe
    ct = ct_ref[...]
    cn = jnp.sum(ct * ct, axis=0, keepdims=True)
    en = jnp.sum(e * e, axis=1, keepdims=True)
    d2 = en - 2.0 * jnp.dot(e, ct, preferred_element_type=jnp.float32) + cn
    wgt = jnp.maximum(
        1.0 - jnp.sqrt(jnp.maximum(d2, 0.0)) * (1.0 / KP_EXTENT), 0.0)
    acc = jnp.zeros((_NB, COUT), jnp.float32)
    for m in range(KS):
        wm = wgt[:, m:m + 1]
        am = jnp.sum((gf * wm).reshape(_NB, K, CIN), axis=1)
        acc = acc + jnp.dot(am, w_ref[m], preferred_element_type=jnp.float32)
    o_ref[...] = acc


def _bn_body(x_ref, g_ref, b_ref, o_ref):
    xv = x_ref[...]
    mu = jnp.mean(xv, axis=0, keepdims=True)
    xc = xv - mu
    var = jnp.mean(xc * xc, axis=0, keepdims=True)
    y = xc * lax.rsqrt(var + 1e-5) * g_ref[...] + b_ref[...]
    o_ref[...] = jnp.maximum(y, 0.0)


def _tc_conv(G, P, Ct, W):
    grid = N // _NB
    return pl.pallas_call(
        _conv_body,
        grid=(grid,),
        in_specs=[
            pl.BlockSpec((_NB * K, D), lambda i: (i, 0)),
            pl.BlockSpec((_NB, 16), lambda i: (i, 0)),
            pl.BlockSpec((16, 16), lambda i: (0, 0)),
            pl.BlockSpec((KS, CIN, COUT), lambda i: (0, 0, 0)),
        ],
        out_specs=pl.BlockSpec((_NB, COUT), lambda i: (i, 0)),
        out_shape=jax.ShapeDtypeStruct((N, COUT), jnp.float32),
    )(G, P, Ct, W)


def _tc_bn(x1, gamma, beta):
    return pl.pallas_call(
        _bn_body,
        out_shape=jax.ShapeDtypeStruct((N, COUT), jnp.float32),
    )(x1, gamma, beta)


def kernel(x, pxyz, pknn, kernel_points, weights, gamma, beta):
    feats = x[0, :, 0, :].T
    ppad = jnp.pad(pxyz[0], ((0, 0), (0, 13)))
    table = jnp.concatenate(
        [feats, ppad, jnp.zeros((N, D - CIN - 16), jnp.float32)], axis=1)
    idx = pknn[0].reshape(-1).astype(jnp.int32)
    G = _sc_gather(idx, table)
    ct = jnp.pad(kernel_points.T, ((0, 13), (0, 1)))
    x1 = _tc_conv(G, ppad, ct, weights)
    y = _tc_bn(x1, gamma[None, :], beta[None, :])
    return y.T.reshape(1, COUT, 1, N)


# speedup vs baseline: 3.3449x; 2.5750x over previous
import functools

import jax
import jax.numpy as jnp
from jax import lax
from jax.experimental import pallas as pl
from jax.experimental.pallas import tpu as pltpu
from jax.experimental.pallas import tpu_sc as plsc

KP_EXTENT = 1.2
KS = 15
CIN = 128
COUT = 128
K = 32
N = 10000
NE = N * K

_CHUNK = 80
_NB = 400
_GB = 8


def _sc_gather(idx, feats, px, py, pz):
    info = plsc.get_sparse_core_info()
    nw = info.num_cores * info.num_subcores
    per_w = NE // nw
    steps = per_w // _CHUNK
    grps = _CHUNK // 16
    mesh = plsc.VectorSubcoreMesh(core_axis_name="c", subcore_axis_name="s")

    @functools.partial(
        pl.kernel,
        mesh=mesh,
        out_type=(
            jax.ShapeDtypeStruct((NE, CIN), jnp.float32),
            jax.ShapeDtypeStruct((NE,), jnp.float32),
            jax.ShapeDtypeStruct((NE,), jnp.float32),
            jax.ShapeDtypeStruct((NE,), jnp.float32),
        ),
        scratch_types=[
            pltpu.VMEM((_CHUNK,), jnp.int32),
            pltpu.VMEM((_CHUNK, CIN), jnp.float32),
            pltpu.VMEM((_CHUNK,), jnp.float32),
            pltpu.VMEM((_CHUNK,), jnp.float32),
            pltpu.VMEM((_CHUNK,), jnp.float32),
            pltpu.SemaphoreType.DMA,
            pltpu.SemaphoreType.DMA,
            pltpu.SemaphoreType.DMA,
            pltpu.SemaphoreType.DMA,
        ],
    )
    def gather_k(idx_hbm, feat_hbm, px_hbm, py_hbm, pz_hbm,
                 gf_hbm, nx_hbm, ny_hbm, nz_hbm,
                 idx_v, rows_v, nx_v, ny_v, nz_v, sem, semx, semy, semz):
        wid = lax.axis_index("s") * info.num_cores + lax.axis_index("c")

        def body(i, carry):
            base = wid * per_w + i * _CHUNK
            pltpu.sync_copy(idx_hbm.at[pl.ds(base, _CHUNK)], idx_v)
            cp = pltpu.async_copy(feat_hbm.at[idx_v], rows_v, sem)
            cx = pltpu.async_copy(px_hbm.at[idx_v], nx_v, semx)
            cy = pltpu.async_copy(py_hbm.at[idx_v], ny_v, semy)
            cz = pltpu.async_copy(pz_hbm.at[idx_v], nz_v, semz)
            cx.wait()
            cy.wait()
            cz.wait()
            cp.wait()
            pltpu.sync_copy(rows_v, gf_hbm.at[pl.ds(base, _CHUNK)])
            pltpu.sync_copy(nx_v, nx_hbm.at[pl.ds(base, _CHUNK)])
            pltpu.sync_copy(ny_v, ny_hbm.at[pl.ds(base, _CHUNK)])
            pltpu.sync_copy(nz_v, nz_hbm.at[pl.ds(base, _CHUNK)])
            return carry

        lax.fori_loop(0, steps, body, 0)

    return gather_k(idx, feats, px, py, pz)


def _conv_body(gf_ref, n3_ref, q3_ref, ct_ref, w_ref, o_ref):
    gf = gf_ref[...]
    ct = ct_ref[...]
    e3 = n3_ref[...] - q3_ref[...]
    dot = jnp.dot(ct, e3, preferred_element_type=jnp.float32)
    en = jnp.sum(e3 * e3, axis=0, keepdims=True)
    cn = jnp.sum(ct * ct, axis=1, keepdims=True)
    d2 = en - 2.0 * dot + cn
    mmask = (lax.broadcasted_iota(jnp.int32, (16, 1), 0) < KS).astype(
        jnp.float32)
    wgt = jnp.maximum(
        1.0 - jnp.sqrt(jnp.maximum(d2, 0.0)) * (1.0 / KP_EXTENT),
        0.0) * mmask
    ew = _GB * K
    bd = (lax.broadcasted_iota(jnp.int32, (16 * _GB, ew), 0) % _GB
          == lax.broadcasted_iota(jnp.int32, (16 * _GB, ew), 1) // K
          ).astype(jnp.float32)
    rep = (lax.broadcasted_iota(jnp.int32, (16 * _GB, 16), 0) // _GB
           == lax.broadcasted_iota(jnp.int32, (16 * _GB, 16), 1)
           ).astype(jnp.float32)
    parts = []
    for g in range(_NB // _GB):
        ws = wgt[:, g * ew:(g + 1) * ew]
        lhs = jnp.dot(rep, ws,
                      preferred_element_type=jnp.float32) * bd
        parts.append(jnp.dot(lhs, gf[g * ew:(g + 1) * ew, :],
                             preferred_element_type=jnp.float32))
    r4 = jnp.concatenate(parts, axis=0).reshape(_NB // _GB, 16, _GB, CIN)
    acc = jnp.zeros((_NB, COUT), jnp.float32)
    for m in range(KS):
        am = r4[:, m].reshape(_NB, CIN)
        acc = acc + jnp.dot(am, w_ref[m], preferred_element_type=jnp.float32)
    o_ref[...] = acc


def _bn_body(x_ref, g_ref, b_ref, o_ref):
    xv = x_ref[...]
    mu = jnp.mean(xv, axis=0, keepdims=True)
    xc = xv - mu
    var = jnp.mean(xc * xc, axis=0, keepdims=True)
    y = xc * lax.rsqrt(var + 1e-5) * g_ref[...] + b_ref[...]
    o_ref[...] = jnp.maximum(y, 0.0)


def _tc_conv(G, N3, Q3, Ct, W):
    grid = N // _NB
    return pl.pallas_call(
        _conv_body,
        grid=(grid,),
        in_specs=[
            pl.BlockSpec((_NB * K, CIN), lambda i: (i, 0)),
            pl.BlockSpec((3, _NB * K), lambda i: (0, i)),
            pl.BlockSpec((3, _NB * K), lambda i: (0, i)),
            pl.BlockSpec((16, 3), lambda i: (0, 0)),
            pl.BlockSpec((KS, CIN, COUT), lambda i: (0, 0, 0)),
        ],
        out_specs=pl.BlockSpec((_NB, COUT), lambda i: (i, 0)),
        out_shape=jax.ShapeDtypeStruct((N, COUT), jnp.float32),
    )(G, N3, Q3, Ct, W)


def _tc_bn(x1, gamma, beta):
    return pl.pallas_call(
        _bn_body,
        out_shape=jax.ShapeDtypeStruct((N, COUT), jnp.float32),
    )(x1, gamma, beta)


def kernel(x, pxyz, pknn, kernel_points, weights, gamma, beta):
    feats = x[0, :, 0, :].T
    p = pxyz[0]
    idx = pknn[0].reshape(-1).astype(jnp.int32)
    G, nx, ny, nz = _sc_gather(idx, feats, p[:, 0], p[:, 1], p[:, 2])
    N3 = jnp.concatenate([nx[None, :], ny[None, :], nz[None, :]], axis=0)
    Q3 = jnp.repeat(p.T, K, axis=1)
    ct = jnp.pad(kernel_points, ((0, 1), (0, 0)))
    x1 = _tc_conv(G, N3, Q3, ct, weights)
    y = _tc_bn(x1, gamma[None, :], beta[None, :])
    return y.T.reshape(1, COUT, 1, N)


# speedup vs baseline: 4.4106x; 1.3186x over previous
import functools

import jax
import jax.numpy as jnp
from jax import lax
from jax.experimental import pallas as pl
from jax.experimental.pallas import tpu as pltpu
from jax.experimental.pallas import tpu_sc as plsc

KP_EXTENT = 1.2
KS = 15
CIN = 128
COUT = 128
K = 32
N = 10000
NE = N * K

_CHUNK = 80
_NB = 400
_GB = 8


def _sc_gather(idx, feats, px, py, pz):
    info = plsc.get_sparse_core_info()
    nw = info.num_cores * info.num_subcores
    per_w = NE // nw
    steps = per_w // _CHUNK
    grps = _CHUNK // 16
    mesh = plsc.VectorSubcoreMesh(core_axis_name="c", subcore_axis_name="s")

    @functools.partial(
        pl.kernel,
        mesh=mesh,
        out_type=(
            jax.ShapeDtypeStruct((NE, CIN), jnp.float32),
            jax.ShapeDtypeStruct((NE,), jnp.float32),
            jax.ShapeDtypeStruct((NE,), jnp.float32),
            jax.ShapeDtypeStruct((NE,), jnp.float32),
        ),
        scratch_types=[
            pltpu.VMEM((2, _CHUNK), jnp.int32),
            pltpu.VMEM((2 * _CHUNK, CIN), jnp.float32),
            pltpu.VMEM((2, _CHUNK), jnp.float32),
            pltpu.VMEM((2, _CHUNK), jnp.float32),
            pltpu.VMEM((2, _CHUNK), jnp.float32),
            pltpu.SemaphoreType.DMA,
            pltpu.SemaphoreType.DMA,
            pltpu.SemaphoreType.DMA,
            pltpu.SemaphoreType.DMA,
            pltpu.SemaphoreType.DMA,
            pltpu.SemaphoreType.DMA,
            pltpu.SemaphoreType.DMA,
            pltpu.SemaphoreType.DMA,
        ],
    )
    def gather_k(idx_hbm, feat_hbm, px_hbm, py_hbm, pz_hbm,
                 gf_hbm, nx_hbm, ny_hbm, nz_hbm,
                 idx_v, rows_v, nx_v, ny_v, nz_v, *sems):
        wid = lax.axis_index("s") * info.num_cores + lax.axis_index("c")

        def bufs(b):
            return (idx_v.at[b], rows_v.at[pl.ds(b * _CHUNK, _CHUNK)],
                    nx_v.at[b], ny_v.at[b], nz_v.at[b],
                    sems[4 * b], sems[4 * b + 1], sems[4 * b + 2],
                    sems[4 * b + 3])

        def issue(b, c):
            iv, rv, xv, yv, zv, s0, s1, s2, s3 = bufs(b)
            base = wid * per_w + c * _CHUNK
            pltpu.sync_copy(idx_hbm.at[pl.ds(base, _CHUNK)], iv)
            pltpu.async_copy(feat_hbm.at[iv], rv, s0)
            pltpu.async_copy(px_hbm.at[iv], xv, s1)
            pltpu.async_copy(py_hbm.at[iv], yv, s2)
            pltpu.async_copy(pz_hbm.at[iv], zv, s3)

        def drain(b, c):
            iv, rv, xv, yv, zv, s0, s1, s2, s3 = bufs(b)
            base = wid * per_w + c * _CHUNK
            pltpu.make_async_copy(feat_hbm.at[iv], rv, s0).wait()
            pltpu.make_async_copy(px_hbm.at[iv], xv, s1).wait()
            pltpu.make_async_copy(py_hbm.at[iv], yv, s2).wait()
            pltpu.make_async_copy(pz_hbm.at[iv], zv, s3).wait()
            pltpu.sync_copy(rv, gf_hbm.at[pl.ds(base, _CHUNK)])
            pltpu.sync_copy(xv, nx_hbm.at[pl.ds(base, _CHUNK)])
            pltpu.sync_copy(yv, ny_hbm.at[pl.ds(base, _CHUNK)])
            pltpu.sync_copy(zv, nz_hbm.at[pl.ds(base, _CHUNK)])

        issue(0, 0)

        def body(g, carry):
            issue(1, 2 * g + 1)
            drain(0, 2 * g)
            issue(0, 2 * g + 2)
            drain(1, 2 * g + 1)
            return carry

        lax.fori_loop(0, (steps - 1) // 2, body, 0)
        drain(0, steps - 1)

    return gather_k(idx, feats, px, py, pz)


def _conv_body(gf_ref, n3_ref, q3_ref, ct_ref, w_ref, o_ref):
    gf = gf_ref[...]
    ct = ct_ref[...]
    e3 = n3_ref[...] - q3_ref[...]
    dot = jnp.dot(ct, e3, preferred_element_type=jnp.float32)
    en = jnp.sum(e3 * e3, axis=0, keepdims=True)
    cn = jnp.sum(ct * ct, axis=1, keepdims=True)
    d2 = en - 2.0 * dot + cn
    mmask = (lax.broadcasted_iota(jnp.int32, (16, 1), 0) < KS).astype(
        jnp.float32)
    wgt = jnp.maximum(
        1.0 - jnp.sqrt(jnp.maximum(d2, 0.0)) * (1.0 / KP_EXTENT),
        0.0) * mmask
    ew = _GB * K
    bd = (lax.broadcasted_iota(jnp.int32, (16 * _GB, ew), 0) % _GB
          == lax.broadcasted_iota(jnp.int32, (16 * _GB, ew), 1) // K
          ).astype(jnp.float32)
    rep = (lax.broadcasted_iota(jnp.int32, (16 * _GB, 16), 0) // _GB
           == lax.broadcasted_iota(jnp.int32, (16 * _GB, 16), 1)
           ).astype(jnp.float32)
    parts = []
    for g in range(_NB // _GB):
        ws = wgt[:, g * ew:(g + 1) * ew]
        lhs = jnp.dot(rep, ws,
                      preferred_element_type=jnp.float32) * bd
        parts.append(jnp.dot(lhs, gf[g * ew:(g + 1) * ew, :],
                             preferred_element_type=jnp.float32))
    r4 = jnp.concatenate(parts, axis=0).reshape(_NB // _GB, 16, _GB, CIN)
    acc = jnp.zeros((_NB, COUT), jnp.float32)
    for m in range(KS):
        am = r4[:, m].reshape(_NB, CIN)
        acc = acc + jnp.dot(am, w_ref[m], preferred_element_type=jnp.float32)
    o_ref[...] = acc


def _bn_body(x_ref, g_ref, b_ref, o_ref):
    xv = x_ref[...]
    mu = jnp.mean(xv, axis=0, keepdims=True)
    xc = xv - mu
    var = jnp.mean(xc * xc, axis=0, keepdims=True)
    y = xc * lax.rsqrt(var + 1e-5) * g_ref[...] + b_ref[...]
    o_ref[...] = jnp.maximum(y, 0.0)


def _tc_conv(G, N3, Q3, Ct, W):
    grid = N // _NB
    return pl.pallas_call(
        _conv_body,
        grid=(grid,),
        in_specs=[
            pl.BlockSpec((_NB * K, CIN), lambda i: (i, 0)),
            pl.BlockSpec((3, _NB * K), lambda i: (0, i)),
            pl.BlockSpec((3, _NB * K), lambda i: (0, i)),
            pl.BlockSpec((16, 3), lambda i: (0, 0)),
            pl.BlockSpec((KS, CIN, COUT), lambda i: (0, 0, 0)),
        ],
        out_specs=pl.BlockSpec((_NB, COUT), lambda i: (i, 0)),
        out_shape=jax.ShapeDtypeStruct((N, COUT), jnp.float32),
    )(G, N3, Q3, Ct, W)


def _tc_bn(x1, gamma, beta):
    return pl.pallas_call(
        _bn_body,
        out_shape=jax.ShapeDtypeStruct((N, COUT), jnp.float32),
    )(x1, gamma, beta)


def kernel(x, pxyz, pknn, kernel_points, weights, gamma, beta):
    feats = x[0, :, 0, :].T
    p = pxyz[0]
    idx = pknn[0].reshape(-1).astype(jnp.int32)
    G, nx, ny, nz = _sc_gather(idx, feats, p[:, 0], p[:, 1], p[:, 2])
    N3 = jnp.concatenate([nx[None, :], ny[None, :], nz[None, :]], axis=0)
    Q3 = jnp.repeat(p.T, K, axis=1)
    ct = jnp.pad(kernel_points, ((0, 1), (0, 0)))
    x1 = _tc_conv(G, N3, Q3, ct, weights)
    y = _tc_bn(x1, gamma[None, :], beta[None, :])
    return y.T.reshape(1, COUT, 1, N)
